# Initial kernel scaffold; baseline (speedup 1.0000x reference)
import jax
import jax.numpy as jnp
from jax.experimental import pallas as pl


def kernel(descriptions, amounts, table, W, b):
    raise NotImplementedError("write your pallas kernel here")


# speedup vs baseline: 2.6173x; 2.6173x over previous
import functools

import jax
import jax.numpy as jnp
from jax import lax
from jax.experimental import pallas as pl
from jax.experimental.pallas import tpu as pltpu
from jax.experimental.pallas import tpu_sc as plsc

B = 16384
L = 50
EMBED = 32
NUM_CAT = 128

NC = 2
NS = 16
NW = NC * NS
RW = B // NW
R = 16
NCHUNK = RW // R
CH = R * L
CHP = 896
_SUBS = [(o, min(128, CH - o)) for o in range(0, CH, 128)]


def _sc_pool_body(desc_hbm, table_hbm, out_hbm, idx_v, gath_v, out_v, sem0, sem1):
  wid = lax.axis_index("s") * NC + lax.axis_index("c")
  row0 = wid * RW
  sems = (sem0, sem1)

  def load_idx(b, k):
    start = (wid * NCHUNK + k) * CHP
    pltpu.sync_copy(desc_hbm.at[pl.ds(start, CHP)], idx_v.at[b])

  def fire_gathers(b, k):
    for (o, sz) in _SUBS:
      pltpu.async_copy(
          table_hbm.at[idx_v.at[b, pl.ds(o, sz)]],
          gath_v.at[b, pl.ds(o, sz), :],
          sems[b],
      )

  def wait_gathers(b):
    for (o, sz) in _SUBS:
      pltpu.make_async_copy(
          table_hbm.at[idx_v.at[b, pl.ds(o, sz)]],
          gath_v.at[b, pl.ds(o, sz), :],
          sems[b],
      ).wait()

  def reduce_chunk(b, k):
    zero = jnp.zeros((16,), jnp.float32)
    for r in range(R):
      def jbody(j, carry, base=r * L):
        a0, a1 = carry
        row = base + j
        return (a0 + gath_v[b, row, pl.ds(0, 16)],
                a1 + gath_v[b, row, pl.ds(16, 16)])
      a0, a1 = lax.fori_loop(0, L, jbody, (zero, zero))
      out_v[b, r, pl.ds(0, 16)] = a0
      out_v[b, r, pl.ds(16, 16)] = a1
    pltpu.sync_copy(out_v.at[b], out_hbm.at[pl.ds(row0 + k * R, R), :])

  load_idx(0, 0)
  fire_gathers(0, 0)

  def body(m, carry):
    for b in (0, 1):
      k = 2 * m + b
      kn = k + 1

      @pl.when(kn < NCHUNK)
      def _():
        load_idx(1 - b, kn)
        fire_gathers(1 - b, kn)

      wait_gathers(b)
      reduce_chunk(b, k)
    return carry

  lax.fori_loop(0, NCHUNK // 2, body, 0)


@functools.partial(
    pl.kernel,
    out_type=jax.ShapeDtypeStruct((B, EMBED), jnp.float32),
    mesh=plsc.VectorSubcoreMesh(core_axis_name="c", subcore_axis_name="s"),
    compiler_params=pltpu.CompilerParams(use_tc_tiling_on_sc=False),
    scratch_types=[
        pltpu.VMEM((2, CHP), jnp.int32),
        pltpu.VMEM((2, CH, EMBED), jnp.float32),
        pltpu.VMEM((2, R, EMBED), jnp.float32),
        pltpu.SemaphoreType.DMA,
        pltpu.SemaphoreType.DMA,
    ],
)
def _sc_pool(desc_hbm, table_hbm, out_hbm, idx_v, gath_v, out_v, sem0, sem1):
  _sc_pool_body(desc_hbm, table_hbm, out_hbm, idx_v, gath_v, out_v, sem0, sem1)


def _tc_linear_body(x_ref, amt_ref, w0_ref, w1_ref, b_ref, out_ref):
  y = jnp.dot(x_ref[...], w0_ref[...], preferred_element_type=jnp.float32)
  y = y * (1.0 / L)
  out_ref[...] = y + amt_ref[...] * w1_ref[...] + b_ref[...]


_TC_BLK = 2048


def _tc_linear(pooled, amounts, w0, w1, b2):
  return pl.pallas_call(
      _tc_linear_body,
      grid=(B // _TC_BLK,),
      in_specs=[
          pl.BlockSpec((_TC_BLK, EMBED), lambda i: (i, 0)),
          pl.BlockSpec((_TC_BLK, 1), lambda i: (i, 0)),
          pl.BlockSpec((EMBED, NUM_CAT), lambda i: (0, 0)),
          pl.BlockSpec((1, NUM_CAT), lambda i: (0, 0)),
          pl.BlockSpec((1, NUM_CAT), lambda i: (0, 0)),
      ],
      out_specs=pl.BlockSpec((_TC_BLK, NUM_CAT), lambda i: (i, 0)),
      out_shape=jax.ShapeDtypeStruct((B, NUM_CAT), jnp.float32),
  )(pooled, amounts, w0, w1, b2)


def kernel(descriptions, amounts, table, W, b):
  desc_pad = jnp.pad(
      descriptions.astype(jnp.int32).reshape(B // R, CH),
      ((0, 0), (0, CHP - CH))).reshape(-1)
  pooled = _sc_pool(desc_pad, table)
  return _tc_linear(pooled, amounts, W[:EMBED], W[EMBED:EMBED + 1],
                    b.reshape(1, NUM_CAT))


# speedup vs baseline: 2.9163x; 1.1143x over previous
import functools

import jax
import jax.numpy as jnp
from jax import lax
from jax.experimental import pallas as pl
from jax.experimental.pallas import tpu as pltpu
from jax.experimental.pallas import tpu_sc as plsc

B = 16384
L = 50
EMBED = 32
NUM_CAT = 128

NC = 2
NS = 16
NW = NC * NS
RW = B // NW
R = 16
NCHUNK = RW // R
CH = R * L
CHP = 896
_SUBS = [(o, min(128, CH - o)) for o in range(0, CH, 128)]

V = 1000000
_CB = 2048
_RB = _CB // 4
_GRID = -(-V // _CB)
VP = _GRID * _CB


def _sc_pool_body(desc_hbm, table_hbm, out_hbm, idx_v, gath_v, out_v, sem0, sem1):
  wid = lax.axis_index("s") * NC + lax.axis_index("c")
  row0 = wid * RW
  sems = (sem0, sem1)

  def load_idx(b, k):
    start = (wid * NCHUNK + k) * CHP
    pltpu.sync_copy(desc_hbm.at[pl.ds(start, CHP)], idx_v.at[b])

  def fire_gathers(b, k):
    for (o, sz) in _SUBS:
      pltpu.async_copy(
          table_hbm.at[idx_v.at[b, pl.ds(o, sz)]],
          gath_v.at[b, pl.ds(o, sz), :],
          sems[b],
      )

  def wait_gathers(b):
    for (o, sz) in _SUBS:
      pltpu.make_async_copy(
          table_hbm.at[idx_v.at[b, pl.ds(o, sz)]],
          gath_v.at[b, pl.ds(o, sz), :],
          sems[b],
      ).wait()

  def reduce_chunk(b, k):
    zero = jnp.zeros((16,), jnp.float32)
    for r in range(R):
      def jbody(j, carry, base=r * L):
        a0, a1 = carry
        row = base + j
        return (a0 + gath_v[b, row, pl.ds(0, 16)],
                a1 + gath_v[b, row, pl.ds(16, 16)])
      a0, a1 = lax.fori_loop(0, L, jbody, (zero, zero))
      out_v[b, r, pl.ds(0, 16)] = a0
      out_v[b, r, pl.ds(16, 16)] = a1
    pltpu.sync_copy(out_v.at[b], out_hbm.at[pl.ds(row0 + k * R, R), :])

  load_idx(0, 0)
  fire_gathers(0, 0)

  def body(m, carry):
    for b in (0, 1):
      k = 2 * m + b
      kn = k + 1

      @pl.when(kn < NCHUNK)
      def _():
        load_idx(1 - b, kn)
        fire_gathers(1 - b, kn)

      wait_gathers(b)
      reduce_chunk(b, k)
    return carry

  lax.fori_loop(0, NCHUNK // 2, body, 0)


@functools.partial(
    pl.kernel,
    out_type=jax.ShapeDtypeStruct((B, EMBED), jnp.float32),
    mesh=plsc.VectorSubcoreMesh(core_axis_name="c", subcore_axis_name="s"),
    compiler_params=pltpu.CompilerParams(use_tc_tiling_on_sc=False),
    scratch_types=[
        pltpu.VMEM((2, CHP), jnp.int32),
        pltpu.VMEM((2, CH, EMBED), jnp.float32),
        pltpu.VMEM((2, R, EMBED), jnp.float32),
        pltpu.SemaphoreType.DMA,
        pltpu.SemaphoreType.DMA,
    ],
)
def _sc_pool(desc_hbm, table_hbm, out_hbm, idx_v, gath_v, out_v, sem0, sem1):
  _sc_pool_body(desc_hbm, table_hbm, out_hbm, idx_v, gath_v, out_v, sem0, sem1)


def _tc_repack_body(xt_ref, out_ref):
  z = jnp.transpose(xt_ref[...])
  out_ref[...] = jnp.concatenate(
      [z[a * _RB:(a + 1) * _RB, :] for a in range(4)], axis=1)


def _tc_repack(tableT):
  return pl.pallas_call(
      _tc_repack_body,
      grid=(_GRID,),
      in_specs=[pl.BlockSpec((EMBED, _CB), lambda i: (0, i))],
      out_specs=pl.BlockSpec((_RB, 128), lambda i: (i, 0)),
      out_shape=jax.ShapeDtypeStruct((_GRID * _RB, 128), jnp.float32),
  )(tableT)


def _tc_linear_body(x_ref, amt_ref, w0_ref, w1_ref, b_ref, out_ref):
  y = jnp.dot(x_ref[...], w0_ref[...], preferred_element_type=jnp.float32)
  y = y * (1.0 / L)
  out_ref[...] = y + amt_ref[...] * w1_ref[...] + b_ref[...]


_TC_BLK = 2048


def _tc_linear(pooled, amounts, w0, w1, b2):
  return pl.pallas_call(
      _tc_linear_body,
      grid=(B // _TC_BLK,),
      in_specs=[
          pl.BlockSpec((_TC_BLK, EMBED), lambda i: (i, 0)),
          pl.BlockSpec((_TC_BLK, 1), lambda i: (i, 0)),
          pl.BlockSpec((EMBED, NUM_CAT), lambda i: (0, 0)),
          pl.BlockSpec((1, NUM_CAT), lambda i: (0, 0)),
          pl.BlockSpec((1, NUM_CAT), lambda i: (0, 0)),
      ],
      out_specs=pl.BlockSpec((_TC_BLK, NUM_CAT), lambda i: (i, 0)),
      out_shape=jax.ShapeDtypeStruct((B, NUM_CAT), jnp.float32),
  )(pooled, amounts, w0, w1, b2)


def kernel(descriptions, amounts, table, W, b):
  dt = descriptions.astype(jnp.int32)
  dt = (dt & -2048) + ((dt & 511) << 2) + ((dt >> 9) & 3)
  desc_pad = jnp.pad(
      dt.reshape(B // R, CH), ((0, 0), (0, CHP - CH))).reshape(-1)
  table_lin = _tc_repack(table.T).reshape(VP, EMBED)
  pooled = _sc_pool(desc_pad, table_lin)
  return _tc_linear(pooled, amounts, W[:EMBED], W[EMBED:EMBED + 1],
                    b.reshape(1, NUM_CAT))


# speedup vs baseline: 3.4623x; 1.1872x over previous
import functools

import jax
import jax.numpy as jnp
from jax import lax
from jax.experimental import pallas as pl
from jax.experimental.pallas import tpu as pltpu
from jax.experimental.pallas import tpu_sc as plsc

B = 16384
L = 50
EMBED = 32
NUM_CAT = 128

NC = 2
NS = 16
NW = NC * NS
RW = B // NW
R = 16
NCHUNK = RW // R
CH = R * L
CHP = 896
_SUBS = [(o, min(128, CH - o)) for o in range(0, CH, 128)]

V = 1000000
_CB = 2048
_RB = _CB // 4
_GRID = -(-V // _CB)
VP = _GRID * _CB


def _sc_pool_body(desc_hbm, table_hbm, out_hbm, idx_v, gath_v, out_v, sem0, sem1):
  wid = lax.axis_index("s") * NC + lax.axis_index("c")
  row0 = wid * RW
  sems = (sem0, sem1)

  def load_idx(b, k):
    start = (wid * NCHUNK + k) * CHP
    pltpu.sync_copy(desc_hbm.at[pl.ds(start, CHP)], idx_v.at[b])

  def fire_gathers(b, k):
    for (o, sz) in _SUBS:
      pltpu.async_copy(
          table_hbm.at[idx_v.at[b, pl.ds(o, sz)]],
          gath_v.at[b, pl.ds(o, sz), :],
          sems[b],
      )

  def wait_gathers(b):
    for (o, sz) in _SUBS:
      pltpu.make_async_copy(
          table_hbm.at[idx_v.at[b, pl.ds(o, sz)]],
          gath_v.at[b, pl.ds(o, sz), :],
          sems[b],
      ).wait()

  def reduce_chunk(b, k):
    zero = jnp.zeros((16,), jnp.float32)
    for r in range(R):
      def jbody(j, carry, base=r * L):
        a0, a1 = carry
        row = base + j
        return (a0 + gath_v[b, row, pl.ds(0, 16)],
                a1 + gath_v[b, row, pl.ds(16, 16)])
      a0, a1 = lax.fori_loop(0, L, jbody, (zero, zero))
      out_v[b, r, pl.ds(0, 16)] = a0
      out_v[b, r, pl.ds(16, 16)] = a1
    pltpu.sync_copy(out_v.at[b], out_hbm.at[pl.ds(row0 + k * R, R), :])

  load_idx(0, 0)
  fire_gathers(0, 0)

  def body(m, carry):
    for b in (0, 1):
      k = 2 * m + b
      kn = k + 1

      @pl.when(kn < NCHUNK)
      def _():
        load_idx(1 - b, kn)
        fire_gathers(1 - b, kn)

      wait_gathers(b)
      reduce_chunk(b, k)
    return carry

  lax.fori_loop(0, NCHUNK // 2, body, 0)


@functools.partial(
    pl.kernel,
    out_type=jax.ShapeDtypeStruct((B, EMBED), jnp.float32),
    mesh=plsc.VectorSubcoreMesh(core_axis_name="c", subcore_axis_name="s"),
    compiler_params=pltpu.CompilerParams(use_tc_tiling_on_sc=False),
    scratch_types=[
        pltpu.VMEM((2, CHP), jnp.int32),
        pltpu.VMEM((2, CH, EMBED), jnp.float32),
        pltpu.VMEM((2, R, EMBED), jnp.float32),
        pltpu.SemaphoreType.DMA,
        pltpu.SemaphoreType.DMA,
    ],
)
def _sc_pool(desc_hbm, table_hbm, out_hbm, idx_v, gath_v, out_v, sem0, sem1):
  _sc_pool_body(desc_hbm, table_hbm, out_hbm, idx_v, gath_v, out_v, sem0, sem1)


def _tc_repack_body(xt_ref, out_ref):
  x = xt_ref[...]
  X = jnp.concatenate(
      [x[:, a * _RB:(a + 1) * _RB] for a in range(4)], axis=0)
  out_ref[...] = jnp.transpose(X)


def _tc_repack(tableT):
  return pl.pallas_call(
      _tc_repack_body,
      grid=(_GRID,),
      in_specs=[pl.BlockSpec((EMBED, _CB), lambda i: (0, i))],
      out_specs=pl.BlockSpec((_RB, 128), lambda i: (i, 0)),
      out_shape=jax.ShapeDtypeStruct((_GRID * _RB, 128), jnp.float32),
  )(tableT)


def _tc_linear_body(x_ref, amt_ref, w0_ref, w1_ref, b_ref, out_ref):
  y = jnp.dot(x_ref[...], w0_ref[...], preferred_element_type=jnp.float32)
  y = y * (1.0 / L)
  out_ref[...] = y + amt_ref[...] * w1_ref[...] + b_ref[...]


_TC_BLK = 2048


def _tc_linear(pooled, amounts, w0, w1, b2):
  return pl.pallas_call(
      _tc_linear_body,
      grid=(B // _TC_BLK,),
      in_specs=[
          pl.BlockSpec((_TC_BLK, EMBED), lambda i: (i, 0)),
          pl.BlockSpec((_TC_BLK, 1), lambda i: (i, 0)),
          pl.BlockSpec((EMBED, NUM_CAT), lambda i: (0, 0)),
          pl.BlockSpec((1, NUM_CAT), lambda i: (0, 0)),
          pl.BlockSpec((1, NUM_CAT), lambda i: (0, 0)),
      ],
      out_specs=pl.BlockSpec((_TC_BLK, NUM_CAT), lambda i: (i, 0)),
      out_shape=jax.ShapeDtypeStruct((B, NUM_CAT), jnp.float32),
  )(pooled, amounts, w0, w1, b2)


def kernel(descriptions, amounts, table, W, b):
  dt = descriptions.astype(jnp.int32)
  dt = (dt & -2048) + ((dt & 511) << 2) + ((dt >> 9) & 3)
  desc_pad = jnp.pad(
      dt.reshape(B // R, CH), ((0, 0), (0, CHP - CH))).reshape(-1)
  table_lin = _tc_repack(table.T).reshape(VP, EMBED)
  pooled = _sc_pool(desc_pad, table_lin)
  return _tc_linear(pooled, amounts, W[:EMBED], W[EMBED:EMBED + 1],
                    b.reshape(1, NUM_CAT))


# speedup vs baseline: 5.5213x; 1.5947x over previous
import functools

import jax
import jax.numpy as jnp
from jax import lax
from jax.experimental import pallas as pl
from jax.experimental.pallas import tpu as pltpu
from jax.experimental.pallas import tpu_sc as plsc

B = 16384
L = 50
EMBED = 32
NUM_CAT = 128

NC = 2
NS = 16
NW = NC * NS
RW = B // NW
R = 16
NCHUNK = RW // R
CH = R * L
CHP = 896
_SUBS = [(o, min(128, CH - o)) for o in range(0, CH, 128)]

V = 1000000
_CB = 8192
_RB = _CB // 4
_GRID = -(-V // _CB)
VP = _GRID * _CB


def _sc_pool_body(desc_hbm, table_hbm, out_hbm, idx_v, gath_v, out_v, sem0, sem1):
  wid = lax.axis_index("s") * NC + lax.axis_index("c")
  row0 = wid * RW
  sems = (sem0, sem1)

  def load_idx(b, k):
    start = (wid * NCHUNK + k) * CHP
    pltpu.sync_copy(desc_hbm.at[pl.ds(start, CHP)], idx_v.at[b])

  def fire_gathers(b, k):
    for (o, sz) in _SUBS:
      pltpu.async_copy(
          table_hbm.at[idx_v.at[b, pl.ds(o, sz)]],
          gath_v.at[b, pl.ds(o, sz), :],
          sems[b],
      )

  def wait_gathers(b):
    for (o, sz) in _SUBS:
      pltpu.make_async_copy(
          table_hbm.at[idx_v.at[b, pl.ds(o, sz)]],
          gath_v.at[b, pl.ds(o, sz), :],
          sems[b],
      ).wait()

  def reduce_chunk(b, k):
    zero = jnp.zeros((16,), jnp.float32)
    for r in range(R):
      def jbody(j, carry, base=r * L):
        a0, a1 = carry
        row = base + j
        return (a0 + gath_v[b, row, pl.ds(0, 16)],
                a1 + gath_v[b, row, pl.ds(16, 16)])
      a0, a1 = lax.fori_loop(0, L, jbody, (zero, zero))
      out_v[b, r, pl.ds(0, 16)] = a0
      out_v[b, r, pl.ds(16, 16)] = a1
    pltpu.sync_copy(out_v.at[b], out_hbm.at[pl.ds(row0 + k * R, R), :])

  load_idx(0, 0)
  fire_gathers(0, 0)

  def body(m, carry):
    for b in (0, 1):
      k = 2 * m + b
      kn = k + 1

      @pl.when(kn < NCHUNK)
      def _():
        load_idx(1 - b, kn)
        fire_gathers(1 - b, kn)

      wait_gathers(b)
      reduce_chunk(b, k)
    return carry

  lax.fori_loop(0, NCHUNK // 2, body, 0)


@functools.partial(
    pl.kernel,
    out_type=jax.ShapeDtypeStruct((B, EMBED), jnp.float32),
    mesh=plsc.VectorSubcoreMesh(core_axis_name="c", subcore_axis_name="s"),
    compiler_params=pltpu.CompilerParams(use_tc_tiling_on_sc=False),
    scratch_types=[
        pltpu.VMEM((2, CHP), jnp.int32),
        pltpu.VMEM((2, CH, EMBED), jnp.float32),
        pltpu.VMEM((2, R, EMBED), jnp.float32),
        pltpu.SemaphoreType.DMA,
        pltpu.SemaphoreType.DMA,
    ],
)
def _sc_pool(desc_hbm, table_hbm, out_hbm, idx_v, gath_v, out_v, sem0, sem1):
  _sc_pool_body(desc_hbm, table_hbm, out_hbm, idx_v, gath_v, out_v, sem0, sem1)


def _tc_repack_body(xt_ref, out_ref):
  x = xt_ref[...]
  X = jnp.concatenate(
      [x[:, a * _RB:(a + 1) * _RB] for a in range(4)], axis=0)
  out_ref[...] = jnp.transpose(X)


def _tc_repack(tableT):
  return pl.pallas_call(
      _tc_repack_body,
      grid=(_GRID,),
      in_specs=[pl.BlockSpec((EMBED, _CB), lambda i: (0, i))],
      out_specs=pl.BlockSpec((_RB, 128), lambda i: (i, 0)),
      out_shape=jax.ShapeDtypeStruct((_GRID * _RB, 128), jnp.float32),
  )(tableT)


def _tc_linear_body(x_ref, amt_ref, w0_ref, w1_ref, b_ref, out_ref):
  y = jnp.dot(x_ref[...], w0_ref[...], preferred_element_type=jnp.float32)
  y = y * (1.0 / L)
  out_ref[...] = y + amt_ref[...] * w1_ref[...] + b_ref[...]


_TC_BLK = 2048


def _tc_linear(pooled, amounts, w0, w1, b2):
  return pl.pallas_call(
      _tc_linear_body,
      grid=(B // _TC_BLK,),
      in_specs=[
          pl.BlockSpec((_TC_BLK, EMBED), lambda i: (i, 0)),
          pl.BlockSpec((_TC_BLK, 1), lambda i: (i, 0)),
          pl.BlockSpec((EMBED, NUM_CAT), lambda i: (0, 0)),
          pl.BlockSpec((1, NUM_CAT), lambda i: (0, 0)),
          pl.BlockSpec((1, NUM_CAT), lambda i: (0, 0)),
      ],
      out_specs=pl.BlockSpec((_TC_BLK, NUM_CAT), lambda i: (i, 0)),
      out_shape=jax.ShapeDtypeStruct((B, NUM_CAT), jnp.float32),
  )(pooled, amounts, w0, w1, b2)


def kernel(descriptions, amounts, table, W, b):
  dt = descriptions.astype(jnp.int32)
  dt = (dt & -_CB) + ((dt & (_RB - 1)) << 2) + ((dt // _RB) & 3)
  desc_pad = jnp.pad(
      dt.reshape(B // R, CH), ((0, 0), (0, CHP - CH))).reshape(-1)
  table_lin = _tc_repack(table.T).reshape(VP, EMBED)
  pooled = _sc_pool(desc_pad, table_lin)
  return _tc_linear(pooled, amounts, W[:EMBED], W[EMBED:EMBED + 1],
                    b.reshape(1, NUM_CAT))


# speedup vs baseline: 5.5346x; 1.0024x over previous
import functools

import jax
import jax.numpy as jnp
from jax import lax
from jax.experimental import pallas as pl
from jax.experimental.pallas import tpu as pltpu
from jax.experimental.pallas import tpu_sc as plsc

B = 16384
L = 50
EMBED = 32
NUM_CAT = 128

NC = 2
NS = 16
NW = NC * NS
RW = B // NW
R = 16
NCHUNK = RW // R
CH = R * L
CHP = 896
_SUBS = [(o, min(128, CH - o)) for o in range(0, CH, 128)]

V = 1000000
_CB = 8192
_S = _CB // 8
_GRID = -(-V // _CB)
VP = _GRID * _CB


def _sc_pool_body(desc_hbm, table_hbm, out_hbm, idx_v, gath_v, out_v, sem0, sem1):
  wid = lax.axis_index("s") * NC + lax.axis_index("c")
  row0 = wid * RW
  sems = (sem0, sem1)

  def load_idx(b, k):
    start = (wid * NCHUNK + k) * CHP
    pltpu.sync_copy(desc_hbm.at[pl.ds(start, CHP)], idx_v.at[b])

  def fire_gathers(b, k):
    for (o, sz) in _SUBS:
      pltpu.async_copy(
          table_hbm.at[idx_v.at[b, pl.ds(o, sz)]],
          gath_v.at[b, pl.ds(o, sz), :],
          sems[b],
      )

  def wait_gathers(b):
    for (o, sz) in _SUBS:
      pltpu.make_async_copy(
          table_hbm.at[idx_v.at[b, pl.ds(o, sz)]],
          gath_v.at[b, pl.ds(o, sz), :],
          sems[b],
      ).wait()

  def reduce_chunk(b, k):
    zero = jnp.zeros((16,), jnp.float32)
    for r in range(R):
      def jbody(j, carry, base=r * L):
        a0, a1 = carry
        g = plsc.bitcast(gath_v[b, base + j, :], jnp.bfloat16)
        pa, pb = plsc.unpack(g, format=plsc.PackFormat.INTERLEAVED,
                             preferred_element_type=jnp.float32)
        return (a0 + pa, a1 + pb)
      a0, a1 = lax.fori_loop(0, L, jbody, (zero, zero))
      out_v[b, r, pl.ds(0, 16)] = a0
      out_v[b, r, pl.ds(16, 16)] = a1
    pltpu.sync_copy(out_v.at[b], out_hbm.at[pl.ds(row0 + k * R, R), :])

  load_idx(0, 0)
  fire_gathers(0, 0)

  def body(m, carry):
    for b in (0, 1):
      k = 2 * m + b
      kn = k + 1

      @pl.when(kn < NCHUNK)
      def _():
        load_idx(1 - b, kn)
        fire_gathers(1 - b, kn)

      wait_gathers(b)
      reduce_chunk(b, k)
    return carry

  lax.fori_loop(0, NCHUNK // 2, body, 0)


@functools.partial(
    pl.kernel,
    out_type=jax.ShapeDtypeStruct((B, EMBED), jnp.float32),
    mesh=plsc.VectorSubcoreMesh(core_axis_name="c", subcore_axis_name="s"),
    compiler_params=pltpu.CompilerParams(use_tc_tiling_on_sc=False,
                                         needs_layout_passes=False),
    scratch_types=[
        pltpu.VMEM((2, CHP), jnp.int32),
        pltpu.VMEM((2, CH, 16), jnp.float32),
        pltpu.VMEM((2, R, EMBED), jnp.float32),
        pltpu.SemaphoreType.DMA,
        pltpu.SemaphoreType.DMA,
    ],
)
def _sc_pool(desc_hbm, table_hbm, out_hbm, idx_v, gath_v, out_v, sem0, sem1):
  _sc_pool_body(desc_hbm, table_hbm, out_hbm, idx_v, gath_v, out_v, sem0, sem1)


def _bf16_bits(y):
  u = lax.bitcast_convert_type(y, jnp.uint32)
  return (u + 0x7FFF + ((u >> 16) & 1)) >> 16


def _tc_repack_body(xt_ref, out_ref):
  x = xt_ref[...]
  X0 = jnp.concatenate(
      [x[0:16, a * _S:(a + 1) * _S] for a in range(8)], axis=0)
  X1 = jnp.concatenate(
      [x[16:32, a * _S:(a + 1) * _S] for a in range(8)], axis=0)
  y0 = jnp.transpose(X0)
  y1 = jnp.transpose(X1)
  packed = _bf16_bits(y0) | (_bf16_bits(y1) << 16)
  out_ref[...] = lax.bitcast_convert_type(packed, jnp.float32)


def _tc_repack(tableT):
  return pl.pallas_call(
      _tc_repack_body,
      grid=(_GRID,),
      in_specs=[pl.BlockSpec((EMBED, _CB), lambda i: (0, i))],
      out_specs=pl.BlockSpec((_S, 128), lambda i: (i, 0)),
      out_shape=jax.ShapeDtypeStruct((_GRID * _S, 128), jnp.float32),
  )(tableT)


def _tc_linear_body(x_ref, amt_ref, w0_ref, w1_ref, b_ref, out_ref):
  y = jnp.dot(x_ref[...], w0_ref[...], preferred_element_type=jnp.float32)
  y = y * (1.0 / L)
  out_ref[...] = y + amt_ref[...] * w1_ref[...] + b_ref[...]


_TC_BLK = 2048


def _tc_linear(pooled, amounts, w0, w1, b2):
  return pl.pallas_call(
      _tc_linear_body,
      grid=(B // _TC_BLK,),
      in_specs=[
          pl.BlockSpec((_TC_BLK, EMBED), lambda i: (i, 0)),
          pl.BlockSpec((_TC_BLK, 1), lambda i: (i, 0)),
          pl.BlockSpec((EMBED, NUM_CAT), lambda i: (0, 0)),
          pl.BlockSpec((1, NUM_CAT), lambda i: (0, 0)),
          pl.BlockSpec((1, NUM_CAT), lambda i: (0, 0)),
      ],
      out_specs=pl.BlockSpec((_TC_BLK, NUM_CAT), lambda i: (i, 0)),
      out_shape=jax.ShapeDtypeStruct((B, NUM_CAT), jnp.float32),
  )(pooled, amounts, w0, w1, b2)


def kernel(descriptions, amounts, table, W, b):
  dt = descriptions.astype(jnp.int32)
  dt = (dt & -_CB) + ((dt & (_S - 1)) << 3) + ((dt // _S) & 7)
  desc_pad = jnp.pad(
      dt.reshape(B // R, CH), ((0, 0), (0, CHP - CH))).reshape(-1)
  table_lin = _tc_repack(table.T).reshape(VP, 16)
  pooled = _sc_pool(desc_pad, table_lin)
  return _tc_linear(pooled, amounts, W[:EMBED], W[EMBED:EMBED + 1],
                    b.reshape(1, NUM_CAT))


# speedup vs baseline: 7.8789x; 1.4236x over previous
import functools

import jax
import jax.numpy as jnp
from jax import lax
from jax.experimental import pallas as pl
from jax.experimental.pallas import tpu as pltpu
from jax.experimental.pallas import tpu_sc as plsc

B = 16384
L = 50
EMBED = 32
NUM_CAT = 128

NC = 2
NS = 16
NW = NC * NS
RW = B // NW
R = 64
NCHUNK = RW // R
CH = R * L
_SUBS = [(o, 128) for o in range(0, CH, 128)]

V = 1000000
_CB = 16384
_S = _CB // 8
_GRID = -(-V // _CB)
VP = _GRID * _CB


def _sc_pool_body(desc_hbm, table_hbm, out_hbm, idx_v, gath_v, out_v, sem0, sem1):
  wid = lax.axis_index("s") * NC + lax.axis_index("c")
  row0 = wid * RW
  sems = (sem0, sem1)

  def load_idx(b, k):
    start = (row0 + k * R) * L
    pltpu.sync_copy(desc_hbm.at[pl.ds(start, CH)], idx_v.at[b])

  def fire_gathers(b, k):
    for (o, sz) in _SUBS:
      pltpu.async_copy(
          table_hbm.at[idx_v.at[b, pl.ds(o, sz)]],
          gath_v.at[b, pl.ds(o, sz), :],
          sems[b],
      )

  def wait_gathers(b):
    for (o, sz) in _SUBS:
      pltpu.make_async_copy(
          table_hbm.at[idx_v.at[b, pl.ds(o, sz)]],
          gath_v.at[b, pl.ds(o, sz), :],
          sems[b],
      ).wait()

  def reduce_chunk(b, k):
    zero = jnp.zeros((16,), jnp.float32)
    for r in range(R):
      def jbody(j, carry, base=r * L):
        a0, a1, c0, c1 = carry
        g = plsc.bitcast(gath_v[b, base + j, :], jnp.bfloat16)
        pa, pb = plsc.unpack(g, format=plsc.PackFormat.INTERLEAVED,
                             preferred_element_type=jnp.float32)
        h = plsc.bitcast(gath_v[b, base + 25 + j, :], jnp.bfloat16)
        pc, pd = plsc.unpack(h, format=plsc.PackFormat.INTERLEAVED,
                             preferred_element_type=jnp.float32)
        return (a0 + pa, a1 + pb, c0 + pc, c1 + pd)
      a0, a1, c0, c1 = lax.fori_loop(0, 25, jbody, (zero, zero, zero, zero))
      out_v[b, r, pl.ds(0, 16)] = a0 + c0
      out_v[b, r, pl.ds(16, 16)] = a1 + c1
    pltpu.sync_copy(out_v.at[b], out_hbm.at[pl.ds(row0 + k * R, R), :])

  load_idx(0, 0)
  fire_gathers(0, 0)

  def body(m, carry):
    for b in (0, 1):
      k = 2 * m + b
      kn = k + 1

      @pl.when(kn < NCHUNK)
      def _():
        load_idx(1 - b, kn)
        fire_gathers(1 - b, kn)

      wait_gathers(b)
      reduce_chunk(b, k)
    return carry

  lax.fori_loop(0, NCHUNK // 2, body, 0)


@functools.partial(
    pl.kernel,
    out_type=jax.ShapeDtypeStruct((B, EMBED), jnp.float32),
    mesh=plsc.VectorSubcoreMesh(core_axis_name="c", subcore_axis_name="s"),
    compiler_params=pltpu.CompilerParams(use_tc_tiling_on_sc=False,
                                         needs_layout_passes=False),
    scratch_types=[
        pltpu.VMEM((2, CH), jnp.int32),
        pltpu.VMEM((2, CH, 16), jnp.float32),
        pltpu.VMEM((2, R, EMBED), jnp.float32),
        pltpu.SemaphoreType.DMA,
        pltpu.SemaphoreType.DMA,
    ],
)
def _sc_pool(desc_hbm, table_hbm, out_hbm, idx_v, gath_v, out_v, sem0, sem1):
  _sc_pool_body(desc_hbm, table_hbm, out_hbm, idx_v, gath_v, out_v, sem0, sem1)


def _bf16_bits(y):
  u = lax.bitcast_convert_type(y, jnp.uint32)
  return (u + 0x7FFF + ((u >> 16) & 1)) >> 16


def _tc_repack_body(xt_ref, out_ref):
  x = xt_ref[...]
  X0 = jnp.concatenate(
      [x[0:16, a * _S:(a + 1) * _S] for a in range(8)], axis=0)
  X1 = jnp.concatenate(
      [x[16:32, a * _S:(a + 1) * _S] for a in range(8)], axis=0)
  y0 = jnp.transpose(X0)
  y1 = jnp.transpose(X1)
  packed = _bf16_bits(y0) | (_bf16_bits(y1) << 16)
  out_ref[...] = lax.bitcast_convert_type(packed, jnp.float32)


def _tc_repack(tableT):
  return pl.pallas_call(
      _tc_repack_body,
      grid=(_GRID,),
      in_specs=[pl.BlockSpec((EMBED, _CB), lambda i: (0, i))],
      out_specs=pl.BlockSpec((_S, 128), lambda i: (i, 0)),
      out_shape=jax.ShapeDtypeStruct((_GRID * _S, 128), jnp.float32),
  )(tableT)


def _tc_linear_body(x_ref, amt_ref, w0_ref, w1_ref, b_ref, out_ref):
  y = jnp.dot(x_ref[...], w0_ref[...], preferred_element_type=jnp.float32)
  y = y * (1.0 / L)
  out_ref[...] = y + amt_ref[...] * w1_ref[...] + b_ref[...]


_TC_BLK = 2048


def _tc_linear(pooled, amounts, w0, w1, b2):
  return pl.pallas_call(
      _tc_linear_body,
      grid=(B // _TC_BLK,),
      in_specs=[
          pl.BlockSpec((_TC_BLK, EMBED), lambda i: (i, 0)),
          pl.BlockSpec((_TC_BLK, 1), lambda i: (i, 0)),
          pl.BlockSpec((EMBED, NUM_CAT), lambda i: (0, 0)),
          pl.BlockSpec((1, NUM_CAT), lambda i: (0, 0)),
          pl.BlockSpec((1, NUM_CAT), lambda i: (0, 0)),
      ],
      out_specs=pl.BlockSpec((_TC_BLK, NUM_CAT), lambda i: (i, 0)),
      out_shape=jax.ShapeDtypeStruct((B, NUM_CAT), jnp.float32),
  )(pooled, amounts, w0, w1, b2)


def kernel(descriptions, amounts, table, W, b):
  dt = descriptions.astype(jnp.int32)
  dt = (dt & -_CB) + ((dt & (_S - 1)) << 3) + ((dt // _S) & 7)
  table_lin = _tc_repack(table.T).reshape(VP, 16)
  pooled = _sc_pool(dt.reshape(-1), table_lin)
  return _tc_linear(pooled, amounts, W[:EMBED], W[EMBED:EMBED + 1],
                    b.reshape(1, NUM_CAT))


# speedup vs baseline: 8.0083x; 1.0164x over previous
import functools

import jax
import jax.numpy as jnp
from jax import lax
from jax.experimental import pallas as pl
from jax.experimental.pallas import tpu as pltpu
from jax.experimental.pallas import tpu_sc as plsc

B = 16384
L = 50
EMBED = 32
NUM_CAT = 128

NC = 2
NS = 16
NW = NC * NS
RW = B // NW
R = 64
NCHUNK = RW // R
CH = R * L
_SUBS = [(o, 128) for o in range(0, CH, 128)]

V = 1000000
_CB = 16384
_S = _CB // 8
_GRID = -(-V // _CB)
VP = _GRID * _CB


def _sc_pool_body(desc_hbm, table_hbm, out_hbm, idx_v, gath_v, out_v, sem0, sem1):
  wid = lax.axis_index("s") * NC + lax.axis_index("c")
  row0 = wid * RW
  sems = (sem0, sem1)

  def load_idx(b, k):
    start = (row0 + k * R) * L
    pltpu.sync_copy(desc_hbm.at[pl.ds(start, CH)], idx_v.at[b])

  def fire_gathers(b, k):
    for (o, sz) in _SUBS:
      pltpu.async_copy(
          table_hbm.at[idx_v.at[b, pl.ds(o, sz)]],
          gath_v.at[b, pl.ds(o, sz), :],
          sems[b],
      )

  def wait_gathers(b):
    for (o, sz) in _SUBS:
      pltpu.make_async_copy(
          table_hbm.at[idx_v.at[b, pl.ds(o, sz)]],
          gath_v.at[b, pl.ds(o, sz), :],
          sems[b],
      ).wait()

  def zero_pad_lanes(_):
    zero = jnp.zeros((16,), jnp.float32)
    def zrow(i, carry):
      for bb in (0, 1):
        for o in range(32, 128, 16):
          out_v[bb, i, pl.ds(o, 16)] = zero
      return carry
    lax.fori_loop(0, R, zrow, 0)

  def reduce_chunk(b, k):
    zero = jnp.zeros((16,), jnp.float32)
    for r in range(R):
      def jbody(j, carry, base=r * L):
        a0, a1, c0, c1 = carry
        g = plsc.bitcast(gath_v[b, base + j, :], jnp.bfloat16)
        pa, pb = plsc.unpack(g, format=plsc.PackFormat.INTERLEAVED,
                             preferred_element_type=jnp.float32)
        h = plsc.bitcast(gath_v[b, base + 25 + j, :], jnp.bfloat16)
        pc, pd = plsc.unpack(h, format=plsc.PackFormat.INTERLEAVED,
                             preferred_element_type=jnp.float32)
        return (a0 + pa, a1 + pb, c0 + pc, c1 + pd)
      a0, a1, c0, c1 = lax.fori_loop(0, 25, jbody, (zero, zero, zero, zero))
      out_v[b, r, pl.ds(0, 16)] = a0 + c0
      out_v[b, r, pl.ds(16, 16)] = a1 + c1
    pltpu.sync_copy(out_v.at[b], out_hbm.at[pl.ds(row0 + k * R, R), :])

  zero_pad_lanes(None)
  load_idx(0, 0)
  fire_gathers(0, 0)

  def body(m, carry):
    for b in (0, 1):
      k = 2 * m + b
      kn = k + 1

      @pl.when(kn < NCHUNK)
      def _():
        load_idx(1 - b, kn)
        fire_gathers(1 - b, kn)

      wait_gathers(b)
      reduce_chunk(b, k)
    return carry

  lax.fori_loop(0, NCHUNK // 2, body, 0)


@functools.partial(
    pl.kernel,
    out_type=jax.ShapeDtypeStruct((B, 128), jnp.float32),
    mesh=plsc.VectorSubcoreMesh(core_axis_name="c", subcore_axis_name="s"),
    compiler_params=pltpu.CompilerParams(use_tc_tiling_on_sc=False,
                                         needs_layout_passes=False),
    scratch_types=[
        pltpu.VMEM((2, CH), jnp.int32),
        pltpu.VMEM((2, CH, 16), jnp.float32),
        pltpu.VMEM((2, R, 128), jnp.float32),
        pltpu.SemaphoreType.DMA,
        pltpu.SemaphoreType.DMA,
    ],
)
def _sc_pool(desc_hbm, table_hbm, out_hbm, idx_v, gath_v, out_v, sem0, sem1):
  _sc_pool_body(desc_hbm, table_hbm, out_hbm, idx_v, gath_v, out_v, sem0, sem1)


def _bf16_bits(y):
  u = lax.bitcast_convert_type(y, jnp.uint32)
  return (u + 0x7FFF + ((u >> 16) & 1)) >> 16


def _tc_repack_body(xt_ref, out_ref):
  x = xt_ref[...]
  X0 = jnp.concatenate(
      [x[0:16, a * _S:(a + 1) * _S] for a in range(8)], axis=0)
  X1 = jnp.concatenate(
      [x[16:32, a * _S:(a + 1) * _S] for a in range(8)], axis=0)
  y0 = jnp.transpose(X0)
  y1 = jnp.transpose(X1)
  packed = _bf16_bits(y0) | (_bf16_bits(y1) << 16)
  out_ref[...] = lax.bitcast_convert_type(packed, jnp.float32)


def _tc_repack(tableT):
  return pl.pallas_call(
      _tc_repack_body,
      grid=(_GRID,),
      in_specs=[pl.BlockSpec((EMBED, _CB), lambda i: (0, i))],
      out_specs=pl.BlockSpec((_S, 128), lambda i: (i, 0)),
      out_shape=jax.ShapeDtypeStruct((_GRID * _S, 128), jnp.float32),
  )(tableT)


def _tc_linear_body(x_ref, amt_ref, w0_ref, w1_ref, b_ref, out_ref):
  y = jnp.dot(x_ref[...], w0_ref[...], preferred_element_type=jnp.float32)
  y = y * (1.0 / L)
  out_ref[...] = y + amt_ref[...] * w1_ref[...] + b_ref[...]


_TC_BLK = 2048


def _tc_linear(pooled, amounts, w0, w1, b2):
  return pl.pallas_call(
      _tc_linear_body,
      grid=(B // _TC_BLK,),
      in_specs=[
          pl.BlockSpec((_TC_BLK, 128), lambda i: (i, 0)),
          pl.BlockSpec((_TC_BLK, 1), lambda i: (i, 0)),
          pl.BlockSpec((128, NUM_CAT), lambda i: (0, 0)),
          pl.BlockSpec((1, NUM_CAT), lambda i: (0, 0)),
          pl.BlockSpec((1, NUM_CAT), lambda i: (0, 0)),
      ],
      out_specs=pl.BlockSpec((_TC_BLK, NUM_CAT), lambda i: (i, 0)),
      out_shape=jax.ShapeDtypeStruct((B, NUM_CAT), jnp.float32),
  )(pooled, amounts, w0, w1, b2)


def kernel(descriptions, amounts, table, W, b):
  dt = descriptions.astype(jnp.int32)
  dt = (dt & -_CB) + ((dt & (_S - 1)) << 3) + ((dt // _S) & 7)
  table_lin = _tc_repack(table.T).reshape(VP, 16)
  pooled = _sc_pool(dt.reshape(-1), table_lin)
  w0 = jnp.pad(W[:EMBED], ((0, 128 - EMBED), (0, 0)))
  return _tc_linear(pooled, amounts, w0, W[EMBED:EMBED + 1],
                    b.reshape(1, NUM_CAT))


# speedup vs baseline: 8.0198x; 1.0014x over previous
import functools

import jax
import jax.numpy as jnp
from jax import lax
from jax.experimental import pallas as pl
from jax.experimental.pallas import tpu as pltpu
from jax.experimental.pallas import tpu_sc as plsc

B = 16384
L = 50
EMBED = 32
NUM_CAT = 128

NC = 2
NS = 16
NW = NC * NS
RW = B // NW
R = 64
NCHUNK = RW // R
CH = R * L
_SUBS = [(o, 128) for o in range(0, CH, 128)]

V = 1000000
_CB = 16384
_S = _CB // 8
_GRID = -(-V // _CB)
VP = _GRID * _CB


def _sc_pool_body(desc_hbm, table_hbm, out_hbm, idx_v, gath_v, out_v, sem0, sem1):
  wid = lax.axis_index("s") * NC + lax.axis_index("c")
  row0 = wid * RW
  sems = (sem0, sem1)

  def load_idx(b, k):
    start = (row0 + k * R) * L
    pltpu.sync_copy(desc_hbm.at[pl.ds(start, CH)], idx_v.at[b])

  def fire_gathers(b, k):
    for (o, sz) in _SUBS:
      pltpu.async_copy(
          table_hbm.at[idx_v.at[b, pl.ds(o, sz)]],
          gath_v.at[b, pl.ds(o, sz), :],
          sems[b],
      )

  def wait_gathers(b):
    for (o, sz) in _SUBS:
      pltpu.make_async_copy(
          table_hbm.at[idx_v.at[b, pl.ds(o, sz)]],
          gath_v.at[b, pl.ds(o, sz), :],
          sems[b],
      ).wait()

  def zero_pad_lanes(_):
    zero = jnp.zeros((16,), jnp.float32)
    def zrow(i, carry):
      for bb in (0, 1):
        for o in range(32, 128, 16):
          out_v[bb, i, pl.ds(o, 16)] = zero
      return carry
    lax.fori_loop(0, R, zrow, 0)

  def reduce_chunk(b, k):
    zero = jnp.zeros((16,), jnp.float32)
    for r in range(R):
      def jbody(j, carry, base=r * L):
        a0, a1, c0, c1 = carry
        u = plsc.bitcast(gath_v[b, base + j, :], jnp.int32)
        w = plsc.bitcast(gath_v[b, base + 25 + j, :], jnp.int32)
        pa = plsc.bitcast(u << 16, jnp.float32)
        pb = plsc.bitcast(u & jnp.int32(-65536), jnp.float32)
        pc = plsc.bitcast(w << 16, jnp.float32)
        pd = plsc.bitcast(w & jnp.int32(-65536), jnp.float32)
        return (a0 + pa, a1 + pb, c0 + pc, c1 + pd)
      a0, a1, c0, c1 = lax.fori_loop(0, 25, jbody, (zero, zero, zero, zero))
      out_v[b, r, pl.ds(0, 16)] = a0 + c0
      out_v[b, r, pl.ds(16, 16)] = a1 + c1
    pltpu.sync_copy(out_v.at[b], out_hbm.at[pl.ds(row0 + k * R, R), :])

  zero_pad_lanes(None)
  load_idx(0, 0)
  fire_gathers(0, 0)

  def body(m, carry):
    for b in (0, 1):
      k = 2 * m + b
      kn = k + 1

      @pl.when(kn < NCHUNK)
      def _():
        load_idx(1 - b, kn)
        fire_gathers(1 - b, kn)

      wait_gathers(b)
      reduce_chunk(b, k)
    return carry

  lax.fori_loop(0, NCHUNK // 2, body, 0)


@functools.partial(
    pl.kernel,
    out_type=jax.ShapeDtypeStruct((B, 128), jnp.float32),
    mesh=plsc.VectorSubcoreMesh(core_axis_name="c", subcore_axis_name="s"),
    compiler_params=pltpu.CompilerParams(use_tc_tiling_on_sc=False,
                                         needs_layout_passes=False),
    scratch_types=[
        pltpu.VMEM((2, CH), jnp.int32),
        pltpu.VMEM((2, CH, 16), jnp.float32),
        pltpu.VMEM((2, R, 128), jnp.float32),
        pltpu.SemaphoreType.DMA,
        pltpu.SemaphoreType.DMA,
    ],
)
def _sc_pool(desc_hbm, table_hbm, out_hbm, idx_v, gath_v, out_v, sem0, sem1):
  _sc_pool_body(desc_hbm, table_hbm, out_hbm, idx_v, gath_v, out_v, sem0, sem1)


def _bf16_bits(y):
  u = lax.bitcast_convert_type(y, jnp.uint32)
  return (u + 0x7FFF + ((u >> 16) & 1)) >> 16


def _tc_repack_body(xt_ref, out_ref):
  x = xt_ref[...]
  X0 = jnp.concatenate(
      [x[0:16, a * _S:(a + 1) * _S] for a in range(8)], axis=0)
  X1 = jnp.concatenate(
      [x[16:32, a * _S:(a + 1) * _S] for a in range(8)], axis=0)
  y0 = jnp.transpose(X0)
  y1 = jnp.transpose(X1)
  packed = _bf16_bits(y0) | (_bf16_bits(y1) << 16)
  out_ref[...] = lax.bitcast_convert_type(packed, jnp.float32)


def _tc_repack(tableT):
  return pl.pallas_call(
      _tc_repack_body,
      grid=(_GRID,),
      in_specs=[pl.BlockSpec((EMBED, _CB), lambda i: (0, i))],
      out_specs=pl.BlockSpec((_S, 128), lambda i: (i, 0)),
      out_shape=jax.ShapeDtypeStruct((_GRID * _S, 128), jnp.float32),
  )(tableT)


def _tc_linear_body(x_ref, amt_ref, w0_ref, w1_ref, b_ref, out_ref):
  y = jnp.dot(x_ref[...], w0_ref[...], preferred_element_type=jnp.float32)
  y = y * (1.0 / L)
  out_ref[...] = y + amt_ref[...] * w1_ref[...] + b_ref[...]


_TC_BLK = 2048


def _tc_linear(pooled, amounts, w0, w1, b2):
  return pl.pallas_call(
      _tc_linear_body,
      grid=(B // _TC_BLK,),
      in_specs=[
          pl.BlockSpec((_TC_BLK, 128), lambda i: (i, 0)),
          pl.BlockSpec((_TC_BLK, 1), lambda i: (i, 0)),
          pl.BlockSpec((128, NUM_CAT), lambda i: (0, 0)),
          pl.BlockSpec((1, NUM_CAT), lambda i: (0, 0)),
          pl.BlockSpec((1, NUM_CAT), lambda i: (0, 0)),
      ],
      out_specs=pl.BlockSpec((_TC_BLK, NUM_CAT), lambda i: (i, 0)),
      out_shape=jax.ShapeDtypeStruct((B, NUM_CAT), jnp.float32),
  )(pooled, amounts, w0, w1, b2)


def kernel(descriptions, amounts, table, W, b):
  dt = descriptions.astype(jnp.int32)
  dt = (dt & -_CB) + ((dt & (_S - 1)) << 3) + ((dt // _S) & 7)
  table_lin = _tc_repack(table.T).reshape(VP, 16)
  pooled = _sc_pool(dt.reshape(-1), table_lin)
  w0 = jnp.pad(W[:EMBED], ((0, 128 - EMBED), (0, 0)))
  return _tc_linear(pooled, amounts, w0, W[EMBED:EMBED + 1],
                    b.reshape(1, NUM_CAT))


# speedup vs baseline: 8.0708x; 1.0064x over previous
import functools

import jax
import jax.numpy as jnp
from jax import lax
from jax.experimental import pallas as pl
from jax.experimental.pallas import tpu as pltpu
from jax.experimental.pallas import tpu_sc as plsc

B = 16384
L = 50
EMBED = 32
NUM_CAT = 128

NC = 2
NS = 16
NW = NC * NS
RW = B // NW
R = 64
NCHUNK = RW // R
CH = R * L
_SUBS = [(o, 128) for o in range(0, CH, 128)]

V = 1000000
_CB = 16384
_S = _CB // 8
_GRID = -(-V // _CB)
VP = _GRID * _CB


def _sc_pool_body(desc_hbm, table_hbm, out_hbm, idx_v, gath_v, out_v, sem0, sem1):
  wid = lax.axis_index("s") * NC + lax.axis_index("c")
  row0 = wid * RW
  sems = (sem0, sem1)

  def load_idx(b, k):
    start = (row0 + k * R) * L
    pltpu.sync_copy(desc_hbm.at[pl.ds(start, CH)], idx_v.at[b])

  def fire_gathers(b, k):
    for (o, sz) in _SUBS:
      pltpu.async_copy(
          table_hbm.at[idx_v.at[b, pl.ds(o, sz)]],
          gath_v.at[b, pl.ds(o, sz), :],
          sems[b],
      )

  def wait_gathers(b):
    for (o, sz) in _SUBS:
      pltpu.make_async_copy(
          table_hbm.at[idx_v.at[b, pl.ds(o, sz)]],
          gath_v.at[b, pl.ds(o, sz), :],
          sems[b],
      ).wait()

  def zero_pad_lanes(_):
    zero = jnp.zeros((16,), jnp.float32)
    def zrow(i, carry):
      for bb in (0, 1):
        for o in range(32, 128, 16):
          out_v[bb, i, pl.ds(o, 16)] = zero
      return carry
    lax.fori_loop(0, R, zrow, 0)

  def reduce_chunk(b, k):
    zero = jnp.zeros((16,), jnp.float32)
    for r in range(R):
      def jbody(j, carry, base=r * L):
        a0, a1, c0, c1 = carry
        u = plsc.bitcast(gath_v[b, base + j, :], jnp.int32)
        w = plsc.bitcast(gath_v[b, base + 25 + j, :], jnp.int32)
        pa = plsc.bitcast(u << 16, jnp.float32)
        pb = plsc.bitcast(u & jnp.int32(-65536), jnp.float32)
        pc = plsc.bitcast(w << 16, jnp.float32)
        pd = plsc.bitcast(w & jnp.int32(-65536), jnp.float32)
        return (a0 + pa, a1 + pb, c0 + pc, c1 + pd)
      a0, a1, c0, c1 = lax.fori_loop(0, 25, jbody, (zero, zero, zero, zero))
      out_v[b, r, pl.ds(0, 16)] = a0 + c0
      out_v[b, r, pl.ds(16, 16)] = a1 + c1
    pltpu.sync_copy(out_v.at[b], out_hbm.at[pl.ds(row0 + k * R, R), :])

  zero_pad_lanes(None)
  load_idx(0, 0)
  fire_gathers(0, 0)

  def body(m, carry):
    for b in (0, 1):
      k = 2 * m + b
      kn = k + 1

      @pl.when(kn < NCHUNK)
      def _():
        load_idx(1 - b, kn)
        fire_gathers(1 - b, kn)

      wait_gathers(b)
      reduce_chunk(b, k)
    return carry

  lax.fori_loop(0, NCHUNK // 2, body, 0)


@functools.partial(
    pl.kernel,
    out_type=jax.ShapeDtypeStruct((B, 128), jnp.float32),
    mesh=plsc.VectorSubcoreMesh(core_axis_name="c", subcore_axis_name="s"),
    compiler_params=pltpu.CompilerParams(use_tc_tiling_on_sc=False,
                                         needs_layout_passes=False),
    scratch_types=[
        pltpu.VMEM((2, CH), jnp.int32),
        pltpu.VMEM((2, CH, 16), jnp.float32),
        pltpu.VMEM((2, R, 128), jnp.float32),
        pltpu.SemaphoreType.DMA,
        pltpu.SemaphoreType.DMA,
    ],
)
def _sc_pool(desc_hbm, table_hbm, out_hbm, idx_v, gath_v, out_v, sem0, sem1):
  _sc_pool_body(desc_hbm, table_hbm, out_hbm, idx_v, gath_v, out_v, sem0, sem1)


def _bf16_bits(y):
  u = lax.bitcast_convert_type(y, jnp.uint32)
  return (u + 0x7FFF + ((u >> 16) & 1)) >> 16


def _tc_repack_body(xt_ref, out_ref):
  x = xt_ref[...]
  X0 = jnp.concatenate(
      [x[0:16, a * _S:(a + 1) * _S] for a in range(8)], axis=0)
  X1 = jnp.concatenate(
      [x[16:32, a * _S:(a + 1) * _S] for a in range(8)], axis=0)
  y0 = jnp.transpose(X0)
  y1 = jnp.transpose(X1)
  packed = _bf16_bits(y0) | (_bf16_bits(y1) << 16)
  out_ref[...] = lax.bitcast_convert_type(packed, jnp.float32)


def _tc_repack(tableT):
  return pl.pallas_call(
      _tc_repack_body,
      grid=(_GRID,),
      in_specs=[pl.BlockSpec((EMBED, _CB), lambda i: (0, i))],
      out_specs=pl.BlockSpec((_S, 128), lambda i: (i, 0)),
      out_shape=jax.ShapeDtypeStruct((_GRID * _S, 128), jnp.float32),
  )(tableT)


def _tc_linear_body(x_ref, amt_ref, w0_ref, w1_ref, b_ref, out_ref):
  y = jnp.dot(x_ref[...], w0_ref[...], preferred_element_type=jnp.float32)
  y = y * (1.0 / L)
  out_ref[...] = y + amt_ref[...] * w1_ref[...] + b_ref[...]


_TC_BLK = 2048


def _tc_linear(pooled, amounts, w0, w1, b2):
  return pl.pallas_call(
      _tc_linear_body,
      grid=(B // _TC_BLK,),
      in_specs=[
          pl.BlockSpec((_TC_BLK, 128), lambda i: (i, 0)),
          pl.BlockSpec((_TC_BLK, 1), lambda i: (i, 0)),
          pl.BlockSpec((128, NUM_CAT), lambda i: (0, 0)),
          pl.BlockSpec((1, NUM_CAT), lambda i: (0, 0)),
          pl.BlockSpec((1, NUM_CAT), lambda i: (0, 0)),
      ],
      out_specs=pl.BlockSpec((_TC_BLK, NUM_CAT), lambda i: (i, 0)),
      out_shape=jax.ShapeDtypeStruct((B, NUM_CAT), jnp.float32),
  )(pooled, amounts, w0, w1, b2)


def kernel(descriptions, amounts, table, W, b):
  dt = descriptions.astype(jnp.int32).reshape(-1)
  dt = (dt & -_CB) + ((dt & (_S - 1)) << 3) + ((dt // _S) & 7)
  table_lin = _tc_repack(table.T).reshape(VP, 16)
  pooled = _sc_pool(dt, table_lin)
  w0 = jnp.pad(W[:EMBED], ((0, 128 - EMBED), (0, 0)))
  return _tc_linear(pooled, amounts, w0, W[EMBED:EMBED + 1],
                    b.reshape(1, NUM_CAT))


# speedup vs baseline: 8.1594x; 1.0110x over previous
import functools

import jax
import jax.numpy as jnp
from jax import lax
from jax.experimental import pallas as pl
from jax.experimental.pallas import tpu as pltpu
from jax.experimental.pallas import tpu_sc as plsc

B = 16384
L = 50
EMBED = 32
NUM_CAT = 128

NC = 2
NS = 16
NW = NC * NS
RW = B // NW
R = 64
NCHUNK = RW // R
CH = R * L
_SUBS = [(o, 128) for o in range(0, CH, 128)]

V = 1000000
_CB = 16384
_S = _CB // 8
_GRID = -(-V // _CB)
VP = _GRID * _CB


def _sc_pool_body(desc_hbm, amt_hbm, table_hbm, out_hbm, idx_v, gath_v, out_v,
                  amt_v, sem0, sem1):
  wid = lax.axis_index("s") * NC + lax.axis_index("c")
  row0 = wid * RW
  sems = (sem0, sem1)

  def load_idx(b, k):
    start = (row0 + k * R) * L
    pltpu.sync_copy(desc_hbm.at[pl.ds(start, CH)], idx_v.at[b])

  def fire_gathers(b, k):
    for (o, sz) in _SUBS:
      pltpu.async_copy(
          table_hbm.at[idx_v.at[b, pl.ds(o, sz)]],
          gath_v.at[b, pl.ds(o, sz), :],
          sems[b],
      )

  def wait_gathers(b):
    for (o, sz) in _SUBS:
      pltpu.make_async_copy(
          table_hbm.at[idx_v.at[b, pl.ds(o, sz)]],
          gath_v.at[b, pl.ds(o, sz), :],
          sems[b],
      ).wait()

  def zero_pad_lanes(_):
    zero = jnp.zeros((16,), jnp.float32)
    def zrow(i, carry):
      for bb in (0, 1):
        for o in range(32, 128, 16):
          out_v[bb, i, pl.ds(o, 16)] = zero
      return carry
    lax.fori_loop(0, R, zrow, 0)

  def reduce_chunk(b, k):
    zero = jnp.zeros((16,), jnp.float32)
    for r in range(R):
      def jbody(j, carry, base=r * L):
        a0, a1, c0, c1 = carry
        u = plsc.bitcast(gath_v[b, base + j, :], jnp.int32)
        w = plsc.bitcast(gath_v[b, base + 25 + j, :], jnp.int32)
        pa = plsc.bitcast(u << 16, jnp.float32)
        pb = plsc.bitcast(u & jnp.int32(-65536), jnp.float32)
        pc = plsc.bitcast(w << 16, jnp.float32)
        pd = plsc.bitcast(w & jnp.int32(-65536), jnp.float32)
        return (a0 + pa, a1 + pb, c0 + pc, c1 + pd)
      a0, a1, c0, c1 = lax.fori_loop(0, 25, jbody, (zero, zero, zero, zero))
      out_v[b, r, pl.ds(0, 16)] = a0 + c0
      out_v[b, r, pl.ds(16, 16)] = a1 + c1
    for s in range(4):
      av = amt_v[pl.ds(k * R + s * 16, 16)]
      plsc.store_scatter(
          out_v.at[b],
          [s * 16 + lax.iota(jnp.int32, 16), jnp.full((16,), 32, jnp.int32)],
          av)
    pltpu.sync_copy(out_v.at[b], out_hbm.at[pl.ds(row0 + k * R, R), :])

  zero_pad_lanes(None)
  pltpu.sync_copy(amt_hbm.at[pl.ds(row0, RW)], amt_v)
  load_idx(0, 0)
  fire_gathers(0, 0)

  def body(m, carry):
    for b in (0, 1):
      k = 2 * m + b
      kn = k + 1

      @pl.when(kn < NCHUNK)
      def _():
        load_idx(1 - b, kn)
        fire_gathers(1 - b, kn)

      wait_gathers(b)
      reduce_chunk(b, k)
    return carry

  lax.fori_loop(0, NCHUNK // 2, body, 0)


@functools.partial(
    pl.kernel,
    out_type=jax.ShapeDtypeStruct((B, 128), jnp.float32),
    mesh=plsc.VectorSubcoreMesh(core_axis_name="c", subcore_axis_name="s"),
    compiler_params=pltpu.CompilerParams(use_tc_tiling_on_sc=False,
                                         needs_layout_passes=False),
    scratch_types=[
        pltpu.VMEM((2, CH), jnp.int32),
        pltpu.VMEM((2, CH, 16), jnp.float32),
        pltpu.VMEM((2, R, 128), jnp.float32),
        pltpu.VMEM((RW,), jnp.float32),
        pltpu.SemaphoreType.DMA,
        pltpu.SemaphoreType.DMA,
    ],
)
def _sc_pool(desc_hbm, amt_hbm, table_hbm, out_hbm, idx_v, gath_v, out_v,
             amt_v, sem0, sem1):
  _sc_pool_body(desc_hbm, amt_hbm, table_hbm, out_hbm, idx_v, gath_v, out_v,
                amt_v, sem0, sem1)


def _bf16_bits(y):
  u = lax.bitcast_convert_type(y, jnp.uint32)
  return (u + 0x7FFF + ((u >> 16) & 1)) >> 16


def _tc_repack_body(xt_ref, out_ref):
  x = xt_ref[...]
  X0 = jnp.concatenate(
      [x[0:16, a * _S:(a + 1) * _S] for a in range(8)], axis=0)
  X1 = jnp.concatenate(
      [x[16:32, a * _S:(a + 1) * _S] for a in range(8)], axis=0)
  y0 = jnp.transpose(X0)
  y1 = jnp.transpose(X1)
  packed = _bf16_bits(y0) | (_bf16_bits(y1) << 16)
  out_ref[...] = lax.bitcast_convert_type(packed, jnp.float32)


def _tc_repack(tableT):
  return pl.pallas_call(
      _tc_repack_body,
      grid=(_GRID,),
      in_specs=[pl.BlockSpec((EMBED, _CB), lambda i: (0, i))],
      out_specs=pl.BlockSpec((_S, 128), lambda i: (i, 0)),
      out_shape=jax.ShapeDtypeStruct((_GRID * _S, 128), jnp.float32),
  )(tableT)


def _tc_linear_body(x_ref, w0_ref, b_ref, out_ref):
  sv = jnp.where(lax.broadcasted_iota(jnp.int32, (128, 1), 0) < EMBED,
                 1.0 / L, 1.0)
  y = jnp.dot(x_ref[...], w0_ref[...] * sv,
              preferred_element_type=jnp.float32)
  out_ref[...] = y + b_ref[...]


_TC_BLK = 2048


def _tc_linear(pooled, w0, b2):
  return pl.pallas_call(
      _tc_linear_body,
      grid=(B // _TC_BLK,),
      in_specs=[
          pl.BlockSpec((_TC_BLK, 128), lambda i: (i, 0)),
          pl.BlockSpec((128, NUM_CAT), lambda i: (0, 0)),
          pl.BlockSpec((1, NUM_CAT), lambda i: (0, 0)),
      ],
      out_specs=pl.BlockSpec((_TC_BLK, NUM_CAT), lambda i: (i, 0)),
      out_shape=jax.ShapeDtypeStruct((B, NUM_CAT), jnp.float32),
  )(pooled, w0, b2)


def kernel(descriptions, amounts, table, W, b):
  dt = descriptions.astype(jnp.int32).reshape(-1)
  dt = (dt & -_CB) + ((dt & (_S - 1)) << 3) + ((dt // _S) & 7)
  table_lin = _tc_repack(table.T).reshape(VP, 16)
  pooled = _sc_pool(dt, amounts.reshape(-1), table_lin)
  w0 = jnp.pad(W, ((0, 128 - EMBED - 1), (0, 0)))
  return _tc_linear(pooled, w0, b.reshape(1, NUM_CAT))
